# baseline (speedup 1.0000x reference)
import jax
import jax.numpy as jnp
from jax.experimental import pallas as pl

B, K, IN, H, OUT = 8, 256, 128, 64, 128
BPG = 4
ALPHA = 0.2
NEG_BIG = -9000000000000000.0
LOG2E = 1.4426950408889634


def _gat_layer_all(h_all, adj_ref, W_ref, a_ref, ones_col, nh):
    Wh_all = jax.lax.dot_general(h_all, W_ref[...], (((1,), (0,)), ((), ())),
                                 preferred_element_type=jnp.float32)
    a_s = a_ref[...][None, :nh] * LOG2E
    a_d = a_ref[...][None, nh:] * LOG2E
    s_all = jax.lax.dot_general(Wh_all, a_s, (((1,), (1,)), ((), ())),
                                preferred_element_type=jnp.float32)
    d_all = jax.lax.dot_general(a_d, Wh_all, (((1,), (1,)), ((), ())),
                                preferred_element_type=jnp.float32)
    Whx_all = jnp.concatenate([Wh_all, ones_col], axis=1)
    outs = []
    for i in range(BPG):
        lo = i * K
        e = s_all[lo:lo + K] + d_all[:, lo:lo + K]
        e = jnp.maximum(e, ALPHA * e)
        att = jnp.where(adj_ref[i] > 0, e, NEG_BIG)
        m = jnp.max(att, axis=1, keepdims=True)
        p = jnp.exp2(att - m)
        hpx = jax.lax.dot_general(p, Whx_all[lo:lo + K],
                                  (((1,), (0,)), ((), ())),
                                  preferred_element_type=jnp.float32)
        hp = hpx[:, :nh] * (1.0 / hpx[:, nh:nh + 1])
        outs.append(jnp.where(hp > 0, hp,
                              jnp.exp(jnp.minimum(hp, 0.0)) - 1.0))
    return jnp.concatenate(outs, axis=0)


def _gat2_kernel(x_ref, adj_ref, W1_ref, a1_ref, W2_ref, a2_ref, out_ref):
    ones_col = jnp.ones((BPG * K, 1), jnp.float32)
    x_all = x_ref[...].reshape(BPG * K, IN)
    h1_all = _gat_layer_all(x_all, adj_ref, W1_ref, a1_ref, ones_col, H)
    out_all = _gat_layer_all(h1_all, adj_ref, W2_ref, a2_ref, ones_col, OUT)
    out_ref[...] = out_all.reshape(BPG, K, OUT)


def kernel(x, adj, W1, a1, W2, a2):
    out = pl.pallas_call(
        _gat2_kernel,
        grid=(B // BPG,),
        in_specs=[
            pl.BlockSpec((BPG, K, IN), lambda b: (b, 0, 0)),
            pl.BlockSpec((BPG, K, K), lambda b: (b, 0, 0)),
            pl.BlockSpec((IN, H), lambda b: (0, 0)),
            pl.BlockSpec((2 * H,), lambda b: (0,)),
            pl.BlockSpec((H, OUT), lambda b: (0, 0)),
            pl.BlockSpec((2 * OUT,), lambda b: (0,)),
        ],
        out_specs=pl.BlockSpec((BPG, K, OUT), lambda b: (b, 0, 0)),
        out_shape=jax.ShapeDtypeStruct((B, K, OUT), jnp.float32),
    )(x, adj, W1, a1, W2, a2)
    return out


# speedup vs baseline: 1.0327x; 1.0327x over previous
import jax
import jax.numpy as jnp
from jax.experimental import pallas as pl

B, K, IN, H, OUT = 8, 256, 128, 64, 128
ALPHA = 0.2
NEG_BIG = -9000000000000000.0
LOG2E = 1.4426950408889634


def _gat_layer_all(h_all, adj_ref, W_ref, a_ref, ones, nh):
    Wh_all = jax.lax.dot_general(h_all, W_ref[...], (((1,), (0,)), ((), ())),
                                 preferred_element_type=jnp.float32)
    a_s = a_ref[...][None, :nh] * LOG2E
    a_d = a_ref[...][None, nh:] * LOG2E
    s_all = jax.lax.dot_general(Wh_all, a_s, (((1,), (1,)), ((), ())),
                                preferred_element_type=jnp.float32)
    d_all = jax.lax.dot_general(a_d, Wh_all, (((1,), (1,)), ((), ())),
                                preferred_element_type=jnp.float32)
    Whx_all = jnp.concatenate(
        [Wh_all, jnp.ones((B * K, 1), jnp.float32)], axis=1)
    outs = []
    for i in range(B):
        lo = i * K
        e = s_all[lo:lo + K] + d_all[:, lo:lo + K]
        e = jnp.maximum(e, ALPHA * e)
        att = jnp.where(adj_ref[i] > 0, e, NEG_BIG)
        m = jnp.max(att, axis=1, keepdims=True)
        p = jnp.exp2(att - m)
        hpx = jax.lax.dot_general(p, Whx_all[lo:lo + K],
                                  (((1,), (0,)), ((), ())),
                                  preferred_element_type=jnp.float32)
        hp = hpx[:, :nh] * (1.0 / hpx[:, nh:nh + 1])
        outs.append(jnp.where(hp > 0, hp,
                              jnp.exp(jnp.minimum(hp, 0.0)) - 1.0))
    return jnp.concatenate(outs, axis=0)


def _gat2_kernel(x_ref, adj_ref, W1_ref, a1_ref, W2_ref, a2_ref, out_ref):
    ones = jnp.ones((1, K), dtype=jnp.float32)
    x_all = x_ref[...].reshape(B * K, IN)
    h1_all = _gat_layer_all(x_all, adj_ref, W1_ref, a1_ref, ones, H)
    out_all = _gat_layer_all(h1_all, adj_ref, W2_ref, a2_ref, ones, OUT)
    out_ref[...] = out_all.reshape(B, K, OUT)


def kernel(x, adj, W1, a1, W2, a2):
    out = pl.pallas_call(
        _gat2_kernel,
        in_specs=[
            pl.BlockSpec((B, K, IN), lambda: (0, 0, 0)),
            pl.BlockSpec((B, K, K), lambda: (0, 0, 0)),
            pl.BlockSpec((IN, H), lambda: (0, 0)),
            pl.BlockSpec((2 * H,), lambda: (0,)),
            pl.BlockSpec((H, OUT), lambda: (0, 0)),
            pl.BlockSpec((2 * OUT,), lambda: (0,)),
        ],
        out_specs=pl.BlockSpec((B, K, OUT), lambda: (0, 0, 0)),
        out_shape=jax.ShapeDtypeStruct((B, K, OUT), jnp.float32),
    )(x, adj, W1, a1, W2, a2)
    return out
